# baseline (speedup 1.0000x reference)
import functools

import jax
import jax.numpy as jnp
import numpy as np
from jax import lax
from jax.experimental import pallas as pl
from jax.experimental.pallas import tpu as pltpu
from jax.experimental.pallas import tpu_sc as plsc

MAX_LENGTH = 2378
LATENT_SAMPLES = 1600
POS_EMB_DIM = 512
OUTPUT_DIM = 1536
EMBED_DIM = 1024
BATCH = 64

TT = 256
NT_DATA = (LATENT_SAMPLES + TT - 1) // TT
NT = (MAX_LENGTH + TT - 1) // TT
T_PAD = NT * TT
TP_ROWS = NT_DATA * TT

ZR = 64
ZCP = (MAX_LENGTH - TP_ROWS + ZR - 1) // ZR
ZTAIL = MAX_LENGTH - TP_ROWS - (ZCP - 1) * ZR
ZSLOTS = BATCH * ZCP

NW = 32
HALF = BATCH // 2
ROWS = HALF * LATENT_SAMPLES
CH = 80
CPB = LATENT_SAMPLES // CH
SLOTS = HALF * CPB
SPW = SLOTS // NW


def _sinusoidal_pe(seq_length, embedding_dim):
    position = np.arange(seq_length, dtype=np.float32)[:, None]
    div_term = np.exp(
        np.arange(0, embedding_dim, 2, dtype=np.float32)
        * (-np.log(10000.0) / embedding_dim))
    pe = np.zeros((seq_length, embedding_dim), dtype=np.float32)
    pe[:, 0::2] = np.sin(position * div_term)
    pe[:, 1::2] = np.cos(position * div_term)
    return pe


_PE_PAD = _sinusoidal_pe(MAX_LENGTH, POS_EMB_DIM)[:TP_ROWS]


def _sc_gather_body(vq_hbm, codes_hbm, sizes_hbm, out_hbm,
                    idx_v, buf, sizes_v, sem):
    c = lax.axis_index("c")
    s = lax.axis_index("s")
    wid = s * 2 + c
    pltpu.sync_copy(sizes_hbm, sizes_v.at[pl.ds(0, HALF)])

    def body(j, carry):
        g = wid + j * NW
        b = g // CPB
        ch = g - b * CPB
        size_b = sizes_v[pl.ds(b, 16)][0]

        @pl.when(ch * CH < size_b)
        def _():
            off = b * LATENT_SAMPLES + ch * CH
            pltpu.sync_copy(codes_hbm.at[pl.ds(off, CH)], idx_v)
            pltpu.async_copy(vq_hbm.at[idx_v], buf, sem).wait()
            pltpu.sync_copy(buf, out_hbm.at[pl.ds(off, CH)])

        return carry

    lax.fori_loop(0, SPW, body, 0)


def _sc_gather(vq, codes_flat, sizes):
    mesh = plsc.VectorSubcoreMesh(core_axis_name="c", subcore_axis_name="s")
    k = functools.partial(
        pl.kernel,
        out_type=jax.ShapeDtypeStruct((ROWS, EMBED_DIM), jnp.float32),
        mesh=mesh,
        scratch_types=[
            pltpu.VMEM((CH,), jnp.int32),
            pltpu.VMEM((CH, EMBED_DIM), jnp.float32),
            pltpu.VMEM((HALF + 16,), jnp.int32),
            pltpu.SemaphoreType.DMA,
        ],
    )(_sc_gather_body)
    return k(vq, codes_flat, sizes)


def _sc_zerofill_body(zrows_hbm, out_hbm, zbuf, sem):
    c = lax.axis_index("c")
    s = lax.axis_index("s")
    wid = s * 2 + c
    pltpu.sync_copy(zrows_hbm, zbuf)

    def body(j, carry):
        g = wid + j * NW
        b = g // ZCP
        k = g - b * ZCP
        off = (b * MAX_LENGTH + TP_ROWS + k * ZR) * OUTPUT_DIM

        @pl.when(k < ZCP - 1)
        def _():
            pltpu.sync_copy(zbuf, out_hbm.at[pl.ds(off, ZR * OUTPUT_DIM)])

        @pl.when(k == ZCP - 1)
        def _():
            n = ZTAIL * OUTPUT_DIM
            pltpu.sync_copy(zbuf.at[pl.ds(0, n)], out_hbm.at[pl.ds(off, n)])

        return carry

    lax.fori_loop(0, ZSLOTS // NW, body, 0)


def _sc_zerofill(zrows):
    mesh = plsc.VectorSubcoreMesh(core_axis_name="c", subcore_axis_name="s")
    k = functools.partial(
        pl.kernel,
        out_type=jax.ShapeDtypeStruct((BATCH * MAX_LENGTH * OUTPUT_DIM,),
                                      jnp.float32),
        mesh=mesh,
        scratch_types=[
            pltpu.VMEM((ZR * OUTPUT_DIM,), jnp.float32),
            pltpu.SemaphoreType.DMA,
        ],
    )(_sc_zerofill_body)
    return k(zrows).reshape(BATCH, MAX_LENGTH, OUTPUT_DIM)


def _pe_proj_body(pe_ref, w2_ref, b_ref, out_ref):
    acc = jnp.dot(pe_ref[...], w2_ref[...], preferred_element_type=jnp.float32)
    out_ref[...] = acc + b_ref[...]


def _pe_proj(pe_bf, w2t_bf, b_proj):
    return pl.pallas_call(
        _pe_proj_body,
        grid=(NT_DATA,),
        in_specs=[
            pl.BlockSpec((TT, POS_EMB_DIM), lambda t: (t, 0)),
            pl.BlockSpec((POS_EMB_DIM, OUTPUT_DIM), lambda t: (0, 0)),
            pl.BlockSpec((1, OUTPUT_DIM), lambda t: (0, 0)),
        ],
        out_specs=pl.BlockSpec((TT, OUTPUT_DIM), lambda t: (t, 0)),
        out_shape=jax.ShapeDtypeStruct((TP_ROWS, OUTPUT_DIM), jnp.float32),
    )(pe_bf, w2t_bf, b_proj)


def _mask_body(sizes_ref, mask_ref):
    g = pl.program_id(0)
    pos = lax.broadcasted_iota(jnp.int32, (1, T_PAD), 1)
    for i in range(8):
        size = sizes_ref[g * 8 + i]
        mask_ref[pl.ds(i, 1), :] = (pos < size).astype(jnp.int32)


def _mask(sizes):
    grid_spec = pltpu.PrefetchScalarGridSpec(
        num_scalar_prefetch=1,
        grid=(BATCH // 8,),
        in_specs=[],
        out_specs=pl.BlockSpec((8, T_PAD), lambda g, sizes: (g, 0)),
    )
    return pl.pallas_call(
        _mask_body,
        grid_spec=grid_spec,
        out_shape=jax.ShapeDtypeStruct((BATCH, T_PAD), jnp.int32),
    )(sizes)


def _make_main_body(b_off):
    def _main_body(sizes_ref, emb_ref, w_ref, pe_ref, out_prev, out_ref):
        del out_prev
        b = pl.program_id(0)
        t = pl.program_id(1)
        size = sizes_ref[b + b_off]
        base = t * TT

        @pl.when(base < size)
        def _():
            a = emb_ref[0].astype(jnp.bfloat16)
            acc = jnp.dot(a, w_ref[...], preferred_element_type=jnp.float32)
            acc = acc + pe_ref[pl.ds(base, TT), :]
            col_mask = base + lax.broadcasted_iota(jnp.int32, (TT, 1), 0) < size
            out_ref[0] = jnp.where(col_mask, acc, 0.0)

        @pl.when(base >= size)
        def _():
            out_ref[0] = jnp.zeros((TT, OUTPUT_DIM), jnp.float32)

    return _main_body


def _main_half(sizes, emb_g, w1t_bf, pe_proj, out_prev, b_off):
    def _clamp(b, t, sizes):
        return jnp.minimum(t, (sizes[b + b_off] + TT - 1) // TT - 1)

    grid_spec = pltpu.PrefetchScalarGridSpec(
        num_scalar_prefetch=1,
        grid=(HALF, NT_DATA),
        in_specs=[
            pl.BlockSpec((1, TT, EMBED_DIM),
                         lambda b, t, sizes: (b, _clamp(b, t, sizes), 0)),
            pl.BlockSpec((EMBED_DIM, OUTPUT_DIM), lambda b, t, sizes: (0, 0)),
            pl.BlockSpec((TP_ROWS, OUTPUT_DIM), lambda b, t, sizes: (0, 0)),
            pl.BlockSpec(memory_space=pltpu.MemorySpace.HBM),
        ],
        out_specs=pl.BlockSpec(
            (1, TT, OUTPUT_DIM),
            lambda b, t, sizes: (b + b_off, t, 0)),
    )
    return pl.pallas_call(
        _make_main_body(b_off),
        grid_spec=grid_spec,
        out_shape=jax.ShapeDtypeStruct((BATCH, MAX_LENGTH, OUTPUT_DIM),
                                       jnp.float32),
        input_output_aliases={4: 0},
        compiler_params=pltpu.CompilerParams(
            dimension_semantics=("parallel", "arbitrary")),
    )(sizes, emb_g, w1t_bf, pe_proj, out_prev)


def kernel(codes, latent_sizes, vq, W_proj, b_proj):
    codes_flat = codes.reshape(-1).astype(jnp.int32)
    sizes = latent_sizes.reshape(-1).astype(jnp.int32)

    pe_bf = jnp.asarray(_PE_PAD, dtype=jnp.bfloat16)
    w2t_bf = W_proj[:, EMBED_DIM:].T.astype(jnp.bfloat16)
    w1t_bf = W_proj[:, :EMBED_DIM].T.astype(jnp.bfloat16)
    b2d = b_proj.reshape(1, OUTPUT_DIM)

    codes_h = codes_flat.reshape(2, HALF * LATENT_SAMPLES)
    emb0 = _sc_gather(vq, codes_h[0], sizes[:HALF])
    emb1 = _sc_gather(vq, codes_h[1], sizes[HALF:])
    emb0 = emb0.reshape(HALF, LATENT_SAMPLES, EMBED_DIM)
    emb1 = emb1.reshape(HALF, LATENT_SAMPLES, EMBED_DIM)

    pe_proj = _pe_proj(pe_bf, w2t_bf, b2d)
    mask_i = _mask(sizes)
    zrows = jnp.zeros((ZR * OUTPUT_DIM,), jnp.float32)
    out_init = _sc_zerofill(zrows)
    out0 = _main_half(sizes, emb0, w1t_bf, pe_proj, out_init, 0)
    out = _main_half(sizes, emb1, w1t_bf, pe_proj, out0, HALF)

    mask = mask_i[:, :MAX_LENGTH].astype(bool)
    return out, mask


# speedup vs baseline: 1.4760x; 1.4760x over previous
import functools

import jax
import jax.numpy as jnp
import numpy as np
from jax import lax
from jax.experimental import pallas as pl
from jax.experimental.pallas import tpu as pltpu
from jax.experimental.pallas import tpu_sc as plsc

MAX_LENGTH = 2378
LATENT_SAMPLES = 1600
POS_EMB_DIM = 512
OUTPUT_DIM = 1536
EMBED_DIM = 1024
BATCH = 64

TT = 256
NT_DATA = (LATENT_SAMPLES + TT - 1) // TT
NT = (MAX_LENGTH + TT - 1) // TT
T_PAD = NT * TT
TP_ROWS = NT_DATA * TT

ZR = 64
ZTOP = (NT - 1) * TT
ZCP = (ZTOP - TP_ROWS) // ZR
ZSLOTS = BATCH * ZCP

NW = 32
HALF = BATCH // 2
ROWS = HALF * LATENT_SAMPLES
CH = 80
CPB = LATENT_SAMPLES // CH
SLOTS = HALF * CPB
SPW = SLOTS // NW


def _sinusoidal_pe(seq_length, embedding_dim):
    position = np.arange(seq_length, dtype=np.float32)[:, None]
    div_term = np.exp(
        np.arange(0, embedding_dim, 2, dtype=np.float32)
        * (-np.log(10000.0) / embedding_dim))
    pe = np.zeros((seq_length, embedding_dim), dtype=np.float32)
    pe[:, 0::2] = np.sin(position * div_term)
    pe[:, 1::2] = np.cos(position * div_term)
    return pe


_PE_PAD = _sinusoidal_pe(MAX_LENGTH, POS_EMB_DIM)[:TP_ROWS]


def _sc_gather_body(vq_hbm, codes_hbm, sizes_hbm, out_hbm,
                    idx_v, buf, sizes_v, sem):
    c = lax.axis_index("c")
    s = lax.axis_index("s")
    wid = s * 2 + c
    pltpu.sync_copy(sizes_hbm, sizes_v.at[pl.ds(0, HALF)])

    def body(j, carry):
        g = wid + j * NW
        b = g // CPB
        ch = g - b * CPB
        size_b = sizes_v[pl.ds(b, 16)][0]

        @pl.when(ch * CH < size_b)
        def _():
            off = b * LATENT_SAMPLES + ch * CH
            pltpu.sync_copy(codes_hbm.at[pl.ds(off, CH)], idx_v)
            pltpu.async_copy(vq_hbm.at[idx_v], buf, sem).wait()
            pltpu.sync_copy(buf, out_hbm.at[pl.ds(off, CH)])

        return carry

    lax.fori_loop(0, SPW, body, 0)


def _sc_gather(vq, codes_flat, sizes):
    mesh = plsc.VectorSubcoreMesh(core_axis_name="c", subcore_axis_name="s")
    k = functools.partial(
        pl.kernel,
        out_type=jax.ShapeDtypeStruct((ROWS, EMBED_DIM), jnp.float32),
        mesh=mesh,
        scratch_types=[
            pltpu.VMEM((CH,), jnp.int32),
            pltpu.VMEM((CH, EMBED_DIM), jnp.float32),
            pltpu.VMEM((HALF + 16,), jnp.int32),
            pltpu.SemaphoreType.DMA,
        ],
    )(_sc_gather_body)
    return k(vq, codes_flat, sizes)


def _sc_zerofill_body(zrows_hbm, out_hbm, zbuf, sem):
    c = lax.axis_index("c")
    s = lax.axis_index("s")
    wid = s * 2 + c
    pltpu.sync_copy(zrows_hbm, zbuf)

    def body(j, carry):
        g = wid + j * NW
        b = g // ZCP
        k = g - b * ZCP
        row = TP_ROWS + k * ZR
        pltpu.sync_copy(zbuf, out_hbm.at[b, pl.ds(row, ZR)])
        return carry

    lax.fori_loop(0, ZSLOTS // NW, body, 0)


def _sc_zerofill(zrows):
    mesh = plsc.VectorSubcoreMesh(core_axis_name="c", subcore_axis_name="s")
    k = functools.partial(
        pl.kernel,
        out_type=jax.ShapeDtypeStruct((BATCH, MAX_LENGTH, OUTPUT_DIM),
                                      jnp.float32),
        mesh=mesh,
        scratch_types=[
            pltpu.VMEM((ZR, OUTPUT_DIM), jnp.float32),
            pltpu.SemaphoreType.DMA,
        ],
    )(_sc_zerofill_body)
    return k(zrows)


def _pe_proj_body(pe_ref, w2_ref, b_ref, out_ref):
    acc = jnp.dot(pe_ref[...], w2_ref[...], preferred_element_type=jnp.float32)
    out_ref[...] = acc + b_ref[...]


def _pe_proj(pe_bf, w2t_bf, b_proj):
    return pl.pallas_call(
        _pe_proj_body,
        grid=(NT_DATA,),
        in_specs=[
            pl.BlockSpec((TT, POS_EMB_DIM), lambda t: (t, 0)),
            pl.BlockSpec((POS_EMB_DIM, OUTPUT_DIM), lambda t: (0, 0)),
            pl.BlockSpec((1, OUTPUT_DIM), lambda t: (0, 0)),
        ],
        out_specs=pl.BlockSpec((TT, OUTPUT_DIM), lambda t: (t, 0)),
        out_shape=jax.ShapeDtypeStruct((TP_ROWS, OUTPUT_DIM), jnp.float32),
    )(pe_bf, w2t_bf, b_proj)


def _mask_body(sizes_ref, mask_ref):
    g = pl.program_id(0)
    pos = lax.broadcasted_iota(jnp.int32, (1, T_PAD), 1)
    for i in range(8):
        size = sizes_ref[g * 8 + i]
        mask_ref[pl.ds(i, 1), :] = (pos < size).astype(jnp.int32)


def _mask(sizes):
    grid_spec = pltpu.PrefetchScalarGridSpec(
        num_scalar_prefetch=1,
        grid=(BATCH // 8,),
        in_specs=[],
        out_specs=pl.BlockSpec((8, T_PAD), lambda g, sizes: (g, 0)),
    )
    return pl.pallas_call(
        _mask_body,
        grid_spec=grid_spec,
        out_shape=jax.ShapeDtypeStruct((BATCH, T_PAD), jnp.int32),
    )(sizes)


def _make_main_body(b_off):
    def _main_body(sizes_ref, emb_ref, w_ref, pe_ref, out_prev, out_ref):
        del out_prev
        b = pl.program_id(0)
        t = pl.program_id(1)
        size = sizes_ref[b + b_off]
        base = jnp.where(t == NT_DATA, (NT - 1) * TT, t * TT)

        @pl.when(base < size)
        def _():
            a = emb_ref[0].astype(jnp.bfloat16)
            acc = jnp.dot(a, w_ref[...], preferred_element_type=jnp.float32)
            acc = acc + pe_ref[pl.ds(base, TT), :]
            col_mask = base + lax.broadcasted_iota(jnp.int32, (TT, 1), 0) < size
            out_ref[0] = jnp.where(col_mask, acc, 0.0)

        @pl.when(base >= size)
        def _():
            out_ref[0] = jnp.zeros((TT, OUTPUT_DIM), jnp.float32)

    return _main_body


def _main_half(sizes, emb_g, w1t_bf, pe_proj, out_prev, b_off):
    def _clamp(b, t, sizes):
        return jnp.minimum(t, (sizes[b + b_off] + TT - 1) // TT - 1)

    grid_spec = pltpu.PrefetchScalarGridSpec(
        num_scalar_prefetch=1,
        grid=(HALF, NT_DATA + 1),
        in_specs=[
            pl.BlockSpec((1, TT, EMBED_DIM),
                         lambda b, t, sizes: (b, _clamp(b, t, sizes), 0)),
            pl.BlockSpec((EMBED_DIM, OUTPUT_DIM), lambda b, t, sizes: (0, 0)),
            pl.BlockSpec((TP_ROWS, OUTPUT_DIM), lambda b, t, sizes: (0, 0)),
            pl.BlockSpec(memory_space=pltpu.MemorySpace.HBM),
        ],
        out_specs=pl.BlockSpec(
            (1, TT, OUTPUT_DIM),
            lambda b, t, sizes: (
                b + b_off, jnp.where(t < NT_DATA, t, NT - 1), 0)),
    )
    return pl.pallas_call(
        _make_main_body(b_off),
        grid_spec=grid_spec,
        out_shape=jax.ShapeDtypeStruct((BATCH, MAX_LENGTH, OUTPUT_DIM),
                                       jnp.float32),
        input_output_aliases={4: 0},
        compiler_params=pltpu.CompilerParams(
            dimension_semantics=("parallel", "arbitrary")),
    )(sizes, emb_g, w1t_bf, pe_proj, out_prev)


def kernel(codes, latent_sizes, vq, W_proj, b_proj):
    codes_flat = codes.reshape(-1).astype(jnp.int32)
    sizes = latent_sizes.reshape(-1).astype(jnp.int32)

    pe_bf = jnp.asarray(_PE_PAD, dtype=jnp.bfloat16)
    w2t_bf = W_proj[:, EMBED_DIM:].T.astype(jnp.bfloat16)
    w1t_bf = W_proj[:, :EMBED_DIM].T.astype(jnp.bfloat16)
    b2d = b_proj.reshape(1, OUTPUT_DIM)

    codes_h = codes_flat.reshape(2, HALF * LATENT_SAMPLES)
    emb0 = _sc_gather(vq, codes_h[0], sizes[:HALF])
    emb1 = _sc_gather(vq, codes_h[1], sizes[HALF:])
    emb0 = emb0.reshape(HALF, LATENT_SAMPLES, EMBED_DIM)
    emb1 = emb1.reshape(HALF, LATENT_SAMPLES, EMBED_DIM)

    pe_proj = _pe_proj(pe_bf, w2t_bf, b2d)
    mask_i = _mask(sizes)
    zrows = jnp.zeros((ZR, OUTPUT_DIM), jnp.float32)
    out_init = _sc_zerofill(zrows)
    out0 = _main_half(sizes, emb0, w1t_bf, pe_proj, out_init, 0)
    out = _main_half(sizes, emb1, w1t_bf, pe_proj, out0, HALF)

    mask = mask_i[:, :MAX_LENGTH].astype(bool)
    return out, mask
